# baseline (speedup 1.0000x reference)
import functools

import jax
import jax.numpy as jnp
from jax import lax
from jax.experimental import pallas as pl
from jax.experimental.pallas import tpu as pltpu
from jax.experimental.pallas import tpu_sc as plsc

N = 10000
E = 320000
D = 128

NC = 2
NS = 16
NW = NC * NS
LANES = 16

CHUNK = 80
NCHUNKS = E // CHUNK
STEPS = NCHUNKS // NW
NBUF = 3
NSLOT = 6

NPAD = 10240
ROWS_PER_SUB = 624
ZROWS = 52
DEG_PIECE = 2000


def _sc_agg_body(feats_hbm, row1_hbm, col1_hbm, w1_hbm, acc_out, deg_out,
                 col_v0, col_v1, col_v2, col_v3, col_v4, col_v5,
                 row_v0, row_v1, row_v2, row_v3, row_v4, row_v5,
                 w_v0, w_v1, w_v2, w_v3, w_v4, w_v5,
                 rows_v0, rows_v1, rows_v2, zero_v, deg_v, acc_sh, deg_sh,
                 isem0, isem1, isem2, isem3, isem4, isem5,
                 gsem0, gsem1, gsem2, ssem0, ssem1, ssem2):
  cid = lax.axis_index("c")
  sid = lax.axis_index("s")
  wid = sid * NC + cid

  col_v = (col_v0, col_v1, col_v2, col_v3, col_v4, col_v5)
  row_v = (row_v0, row_v1, row_v2, row_v3, row_v4, row_v5)
  w_v = (w_v0, w_v1, w_v2, w_v3, w_v4, w_v5)
  isem = (isem0, isem1, isem2, isem3, isem4, isem5)
  rows_v = (rows_v0, rows_v1, rows_v2)
  gsem = (gsem0, gsem1, gsem2)
  ssem = (ssem0, ssem1, ssem2)

  def idx_copies(t, s):
    base = (t * NW + wid) * CHUNK
    return (
        pltpu.make_async_copy(col1_hbm.at[pl.ds(base, CHUNK)], col_v[s],
                              isem[s]),
        pltpu.make_async_copy(row1_hbm.at[pl.ds(base, CHUNK)], row_v[s],
                              isem[s]),
        pltpu.make_async_copy(w1_hbm.at[pl.ds(base, CHUNK)], w_v[s],
                              isem[s]),
    )

  def start_idx(t, s):
    for cp in idx_copies(t, s):
      cp.start()

  def wait_idx(t, s):
    for cp in idx_copies(t, s):
      cp.wait()

  def start_gather(s, b):
    pltpu.async_copy(feats_hbm.at[col_v[s]], rows_v[b], gsem[b])

  def wait_gather(s, b):
    pltpu.make_async_copy(feats_hbm.at[col_v[s]], rows_v[b],
                          gsem[b]).wait()

  def scale(s, b):
    buf = rows_v[b]

    @pl.loop(0, CHUNK // LANES, unroll=2)
    def _(g):
      wvec = w_v[s][pl.ds(g * LANES, LANES)]
      for e in range(LANES):
        w = wvec[e]
        i = g * LANES + e
        for j in range(D // LANES):
          sl = pl.ds(j * LANES, LANES)
          buf[i, sl] = buf[i, sl] * w

  def start_scatter(s, b):
    pltpu.async_copy(rows_v[b], acc_sh.at[row_v[s]], ssem[b], add=True)
    pltpu.async_copy(w_v[s], deg_sh.at[row_v[s]], ssem[b], add=True)

  def wait_scatter(s, b):
    pltpu.make_async_copy(rows_v[b], acc_sh.at[row_v[s]], ssem[b]).wait()
    pltpu.make_async_copy(w_v[s], deg_sh.at[row_v[s]], ssem[b]).wait()

  for t in range(4):
    start_idx(t, t)

  zeros16 = jnp.zeros((LANES,), jnp.float32)

  @pl.loop(0, ZROWS)
  def _(i):
    for j in range(D // LANES):
      zero_v[i, pl.ds(j * LANES, LANES)] = zeros16

  for m in range(ROWS_PER_SUB // ZROWS):
    pltpu.sync_copy(
        zero_v, acc_sh.at[pl.ds(sid * ROWS_PER_SUB + m * ZROWS, ZROWS)])

  @pl.when(sid == NS - 1)
  def _():
    pltpu.sync_copy(zero_v.at[pl.ds(0, 16)],
                    acc_sh.at[pl.ds(NS * ROWS_PER_SUB, 16)])
  for m in range(NPAD // NS // CHUNK):
    pltpu.sync_copy(
        zero_v.at[0, pl.ds(0, CHUNK)],
        deg_sh.at[pl.ds(sid * (NPAD // NS) + m * CHUNK, CHUNK)])

  wait_idx(0, 0)
  start_gather(0, 0)
  wait_idx(1, 1)
  start_gather(1, 1)
  plsc.subcore_barrier()

  def make_process(b, s):
    def process(t):
      wait_gather(s, b)

      @pl.when(t + 2 < STEPS)
      def _():
        wait_idx(t + 2, (s + 2) % NSLOT)

        @pl.when(t >= 1)
        def _():
          wait_scatter((s + 5) % NSLOT, (b + 2) % NBUF)

        start_gather((s + 2) % NSLOT, (b + 2) % NBUF)

      scale(s, b)
      start_scatter(s, b)

      @pl.when(t + 4 < STEPS)
      def _():
        start_idx(t + 4, (s + 4) % NSLOT)

    return process

  procs = [make_process(t6 % NBUF, t6) for t6 in range(NSLOT)]

  @pl.loop(0, STEPS // NSLOT)
  def _(q):
    t0 = NSLOT * q
    for u in range(NSLOT):
      procs[u](t0 + u)

  for u in range(NSLOT * (STEPS // NSLOT), STEPS):
    procs[u % NSLOT](jnp.int32(u))

  for b in range(NBUF):
    last = STEPS - 1 - ((STEPS - 1 - b) % NBUF)
    wait_scatter(last % NSLOT, b)
  plsc.subcore_barrier()

  pltpu.sync_copy(acc_sh.at[pl.ds(sid * ROWS_PER_SUB, ROWS_PER_SUB)],
                  acc_out.at[cid, pl.ds(sid * ROWS_PER_SUB, ROWS_PER_SUB)])

  @pl.when(sid == NS - 1)
  def _():
    pltpu.sync_copy(acc_sh.at[pl.ds(NS * ROWS_PER_SUB, 16)],
                    acc_out.at[cid, pl.ds(NS * ROWS_PER_SUB, 16)])

  @pl.when(sid == 0)
  def _():
    for m in range(N // DEG_PIECE):
      pltpu.sync_copy(deg_sh.at[pl.ds(m * DEG_PIECE, DEG_PIECE)], deg_v)
      pltpu.sync_copy(deg_v,
                      deg_out.at[pl.ds(cid * N + m * DEG_PIECE, DEG_PIECE)])


_sc_agg = functools.partial(
    pl.kernel,
    out_type=(jax.ShapeDtypeStruct((NC, N, D), jnp.float32),
              jax.ShapeDtypeStruct((NC * N,), jnp.float32)),
    mesh=plsc.VectorSubcoreMesh(core_axis_name="c", subcore_axis_name="s"),
    scratch_types=(
        [pltpu.VMEM((CHUNK,), jnp.int32) for _ in range(NSLOT)]
        + [pltpu.VMEM((CHUNK,), jnp.int32) for _ in range(NSLOT)]
        + [pltpu.VMEM((CHUNK,), jnp.float32) for _ in range(NSLOT)]
        + [pltpu.VMEM((CHUNK, D), jnp.float32) for _ in range(NBUF)]
        + [
            pltpu.VMEM((ZROWS, D), jnp.float32),
            pltpu.VMEM((DEG_PIECE,), jnp.float32),
            pltpu.VMEM_SHARED((N, D), jnp.float32),
            pltpu.VMEM_SHARED((NPAD,), jnp.float32),
        ]
        + [pltpu.SemaphoreType.DMA for _ in range(NSLOT + 2 * NBUF)]
    ),
)(_sc_agg_body)


ROW_BLK = 1000


def _tc_finish_body(acc_ref, deg_ref, feats_ref, wt_ref, b_ref, out_ref):
  s = acc_ref[0] + acc_ref[1]
  d = deg_ref[0] + deg_ref[1]
  h = s / d
  y = jnp.dot(h, wt_ref[...], preferred_element_type=jnp.float32) + b_ref[...]
  out_ref[...] = feats_ref[...] + jnp.maximum(y, 0.0)


def _tc_finish(acc2, deg2, feats, wt, b2):
  return pl.pallas_call(
      _tc_finish_body,
      grid=(N // ROW_BLK,),
      in_specs=[
          pl.BlockSpec((NC, ROW_BLK, D), lambda i: (0, i, 0)),
          pl.BlockSpec((NC, ROW_BLK, 1), lambda i: (0, i, 0)),
          pl.BlockSpec((ROW_BLK, D), lambda i: (i, 0)),
          pl.BlockSpec((D, D), lambda i: (0, 0)),
          pl.BlockSpec((1, D), lambda i: (0, 0)),
      ],
      out_specs=pl.BlockSpec((ROW_BLK, D), lambda i: (i, 0)),
      out_shape=jax.ShapeDtypeStruct((N, D), jnp.float32),
  )(acc2, deg2, feats, wt, b2)


def kernel(feats, edge_index, edge_weight, W, b):
  acc2, deg2 = _sc_agg(feats, edge_index[0], edge_index[1], edge_weight)
  return _tc_finish(acc2, deg2.reshape(NC, N, 1), feats, W.T,
                    b.reshape(1, D))


# speedup vs baseline: 1.1076x; 1.1076x over previous
import functools

import jax
import jax.numpy as jnp
from jax import lax
from jax.experimental import pallas as pl
from jax.experimental.pallas import tpu as pltpu
from jax.experimental.pallas import tpu_sc as plsc

N = 10000
E = 320000
D = 128

NC = 2
NS = 16
NW = NC * NS
LANES = 16

CHUNK = 128
NCHUNKS = E // CHUNK
STEPS = (NCHUNKS + NW - 1) // NW
NBUF = 2
NSLOT = 4
HALF = CHUNK // 2

NPAD = 10240
ROWS_PER_SUB = 624
ZROWS = 52
DEG_PIECE = 2000


def _sc_agg_body(feats_hbm, row1_hbm, col1_hbm, w1_hbm, acc_out, deg_out,
                 col_v0, col_v1, col_v2, col_v3,
                 row_v0, row_v1, row_v2, row_v3,
                 w_v0, w_v1, w_v2, w_v3,
                 rows_v0, rows_v1, zero_v, deg_v, acc_sh, deg_sh,
                 isem0, isem1, isem2, isem3, gsem0, gsem1, ssem0, ssem1):
  cid = lax.axis_index("c")
  sid = lax.axis_index("s")
  wid = sid * NC + cid

  col_v = (col_v0, col_v1, col_v2, col_v3)
  row_v = (row_v0, row_v1, row_v2, row_v3)
  w_v = (w_v0, w_v1, w_v2, w_v3)
  isem = (isem0, isem1, isem2, isem3)
  rows_v = (rows_v0, rows_v1)
  gsem = (gsem0, gsem1)
  ssem = (ssem0, ssem1)

  def idx_copies(t, s):
    base = (t * NW + wid) * CHUNK
    return (
        pltpu.make_async_copy(col1_hbm.at[pl.ds(base, CHUNK)], col_v[s],
                              isem[s]),
        pltpu.make_async_copy(row1_hbm.at[pl.ds(base, CHUNK)], row_v[s],
                              isem[s]),
        pltpu.make_async_copy(w1_hbm.at[pl.ds(base, CHUNK)], w_v[s],
                              isem[s]),
    )

  def start_idx(t, s):
    for cp in idx_copies(t, s):
      cp.start()

  def wait_idx(t, s):
    for cp in idx_copies(t, s):
      cp.wait()

  def valid(t):
    return t * NW + wid < NCHUNKS

  def gather_copies(s, b):
    return (
        pltpu.make_async_copy(feats_hbm.at[col_v[s].at[pl.ds(0, HALF)]],
                              rows_v[b].at[pl.ds(0, HALF)], gsem[b]),
        pltpu.make_async_copy(feats_hbm.at[col_v[s].at[pl.ds(HALF, HALF)]],
                              rows_v[b].at[pl.ds(HALF, HALF)], gsem[b]),
    )

  def start_gather(s, b):
    for cp in gather_copies(s, b):
      cp.start()

  def wait_gather(s, b):
    for cp in gather_copies(s, b):
      cp.wait()

  def scale(s, b):
    buf = rows_v[b]

    @pl.loop(0, CHUNK // LANES, unroll=2)
    def _(g):
      wvec = w_v[s][pl.ds(g * LANES, LANES)]
      for e in range(LANES):
        w = wvec[e]
        i = g * LANES + e
        for j in range(D // LANES):
          sl = pl.ds(j * LANES, LANES)
          buf[i, sl] = buf[i, sl] * w

  def start_scatter(s, b):
    pltpu.async_copy(rows_v[b], acc_sh.at[row_v[s]], ssem[b], add=True)
    pltpu.async_copy(w_v[s], deg_sh.at[row_v[s]], ssem[b], add=True)

  def wait_scatter(s, b):
    pltpu.make_async_copy(rows_v[b], acc_sh.at[row_v[s]], ssem[b]).wait()
    pltpu.make_async_copy(w_v[s], deg_sh.at[row_v[s]], ssem[b]).wait()

  start_idx(0, 0)
  start_idx(1, 1)

  zeros16 = jnp.zeros((LANES,), jnp.float32)

  @pl.loop(0, ZROWS)
  def _(i):
    for j in range(D // LANES):
      zero_v[i, pl.ds(j * LANES, LANES)] = zeros16

  for m in range(ROWS_PER_SUB // ZROWS):
    pltpu.sync_copy(
        zero_v, acc_sh.at[pl.ds(sid * ROWS_PER_SUB + m * ZROWS, ZROWS)])

  @pl.when(sid == NS - 1)
  def _():
    pltpu.sync_copy(zero_v.at[pl.ds(0, 16)],
                    acc_sh.at[pl.ds(NS * ROWS_PER_SUB, 16)])
  for m in range(NPAD // NS // CHUNK):
    pltpu.sync_copy(
        zero_v.at[0],
        deg_sh.at[pl.ds(sid * (NPAD // NS) + m * CHUNK, CHUNK)])

  wait_idx(0, 0)
  start_gather(0, 0)
  plsc.subcore_barrier()

  def make_process(b, s):
    def process(t):
      wait_gather(s, b)

      @pl.when(valid(t + 1))
      def _():
        wait_idx(t + 1, (s + 1) % NSLOT)

        @pl.when(t >= 1)
        def _():
          wait_scatter((s + 3) % NSLOT, 1 - b)

        start_gather((s + 1) % NSLOT, 1 - b)

      scale(s, b)
      start_scatter(s, b)

      @pl.when(valid(t + 2))
      def _():
        start_idx(t + 2, (s + 2) % NSLOT)

    return process

  procs = [make_process(t4 % NBUF, t4) for t4 in range(NSLOT)]

  @pl.loop(0, (STEPS + NSLOT - 1) // NSLOT)
  def _(q):
    t0 = NSLOT * q
    for u in range(NSLOT):
      @pl.when(valid(t0 + u))
      def _(u=u):
        procs[u](t0 + u)

  wait_scatter(0, 0)
  wait_scatter(1, 1)
  plsc.subcore_barrier()

  pltpu.sync_copy(acc_sh.at[pl.ds(sid * ROWS_PER_SUB, ROWS_PER_SUB)],
                  acc_out.at[cid, pl.ds(sid * ROWS_PER_SUB, ROWS_PER_SUB)])

  @pl.when(sid == NS - 1)
  def _():
    pltpu.sync_copy(acc_sh.at[pl.ds(NS * ROWS_PER_SUB, 16)],
                    acc_out.at[cid, pl.ds(NS * ROWS_PER_SUB, 16)])

  @pl.when(sid == 0)
  def _():
    for m in range(N // DEG_PIECE):
      pltpu.sync_copy(deg_sh.at[pl.ds(m * DEG_PIECE, DEG_PIECE)], deg_v)
      pltpu.sync_copy(deg_v,
                      deg_out.at[pl.ds(cid * N + m * DEG_PIECE, DEG_PIECE)])


_sc_agg = functools.partial(
    pl.kernel,
    out_type=(jax.ShapeDtypeStruct((NC, N, D), jnp.float32),
              jax.ShapeDtypeStruct((NC * N,), jnp.float32)),
    mesh=plsc.VectorSubcoreMesh(core_axis_name="c", subcore_axis_name="s"),
    scratch_types=(
        [pltpu.VMEM((CHUNK,), jnp.int32) for _ in range(NSLOT)]
        + [pltpu.VMEM((CHUNK,), jnp.int32) for _ in range(NSLOT)]
        + [pltpu.VMEM((CHUNK,), jnp.float32) for _ in range(NSLOT)]
        + [pltpu.VMEM((CHUNK, D), jnp.float32) for _ in range(NBUF)]
        + [
            pltpu.VMEM((ZROWS, D), jnp.float32),
            pltpu.VMEM((DEG_PIECE,), jnp.float32),
            pltpu.VMEM_SHARED((N, D), jnp.float32),
            pltpu.VMEM_SHARED((NPAD,), jnp.float32),
        ]
        + [pltpu.SemaphoreType.DMA for _ in range(NSLOT + 2 * NBUF)]
    ),
)(_sc_agg_body)


ROW_BLK = 1000


def _tc_finish_body(acc_ref, deg_ref, feats_ref, wt_ref, b_ref, out_ref):
  s = acc_ref[0] + acc_ref[1]
  d = deg_ref[0] + deg_ref[1]
  h = s / d
  y = jnp.dot(h, wt_ref[...], preferred_element_type=jnp.float32) + b_ref[...]
  out_ref[...] = feats_ref[...] + jnp.maximum(y, 0.0)


def _tc_finish(acc2, deg2, feats, wt, b2):
  return pl.pallas_call(
      _tc_finish_body,
      grid=(N // ROW_BLK,),
      in_specs=[
          pl.BlockSpec((NC, ROW_BLK, D), lambda i: (0, i, 0)),
          pl.BlockSpec((NC, ROW_BLK, 1), lambda i: (0, i, 0)),
          pl.BlockSpec((ROW_BLK, D), lambda i: (i, 0)),
          pl.BlockSpec((D, D), lambda i: (0, 0)),
          pl.BlockSpec((1, D), lambda i: (0, 0)),
      ],
      out_specs=pl.BlockSpec((ROW_BLK, D), lambda i: (i, 0)),
      out_shape=jax.ShapeDtypeStruct((N, D), jnp.float32),
  )(acc2, deg2, feats, wt, b2)


def kernel(feats, edge_index, edge_weight, W, b):
  acc2, deg2 = _sc_agg(feats, edge_index[0], edge_index[1], edge_weight)
  return _tc_finish(acc2, deg2.reshape(NC, N, 1), feats, W.T,
                    b.reshape(1, D))
